# Initial kernel scaffold; baseline (speedup 1.0000x reference)
import jax
import jax.numpy as jnp
from jax.experimental import pallas as pl


def kernel(features, relations, edge_index, edge_type, W_neigh, W_loop):
    raise NotImplementedError("write your pallas kernel here")


# speedup vs baseline: 5.3362x; 5.3362x over previous
import functools

import jax
import jax.numpy as jnp
from jax import lax
from jax.experimental import pallas as pl
from jax.experimental.pallas import tpu as pltpu
from jax.experimental.pallas import tpu_sc as plsc

_NC = 2
_NS = 16
_K = 80


def _make_sc_segment_sum(n_rows_table, d, n_edges, n_nodes):
    del n_rows_table
    n_workers = _NC * _NS
    e_per_w = n_edges // n_workers
    n_chunks = e_per_w // _K
    n_pad = ((n_nodes + 8 * _NS - 1) // (8 * _NS)) * (8 * _NS)
    rows_per_tile = n_pad // _NS
    mesh = plsc.VectorSubcoreMesh(core_axis_name="c", subcore_axis_name="s")

    @functools.partial(
        pl.kernel,
        mesh=mesh,
        out_type=jax.ShapeDtypeStruct((_NC * n_pad, d), jnp.float32),
        scratch_types=[
            pltpu.VMEM((n_chunks, _K), jnp.int32),
            pltpu.VMEM((n_chunks, _K), jnp.int32),
            pltpu.VMEM((_K, d), jnp.float32),
            pltpu.VMEM_SHARED((n_pad, d), jnp.float32),
            pltpu.SemaphoreType.DMA,
        ],
        compiler_params=pltpu.CompilerParams(use_tc_tiling_on_sc=False),
    )
    def kern(table_hbm, gidx_hbm, dst_hbm, zeros_hbm, out_hbm,
             gidx_v, didx_v, rows_v, acc, sem):
        cid = lax.axis_index("c")
        sid = lax.axis_index("s")
        wid = cid * _NS + sid
        r0 = sid * rows_per_tile
        pltpu.sync_copy(zeros_hbm, acc.at[pl.ds(r0, rows_per_tile)])
        pltpu.sync_copy(gidx_hbm.at[wid], gidx_v)
        pltpu.sync_copy(dst_hbm.at[wid], didx_v)
        plsc.subcore_barrier()

        def body(c, carry):
            pltpu.async_copy(table_hbm.at[gidx_v.at[c]], rows_v, sem).wait()
            pltpu.sync_copy(rows_v, acc.at[didx_v.at[c]], add=True)
            return carry

        lax.fori_loop(0, n_chunks, body, 0)
        plsc.subcore_barrier()
        pltpu.sync_copy(acc.at[pl.ds(r0, rows_per_tile)],
                        out_hbm.at[pl.ds(cid * n_pad + r0, rows_per_tile)])

    return kern, n_pad


def _make_tc_prologue(n, d_aug, d, bn):
    nb = n // bn

    def body(a0_ref, a1_ref, t_ref, norm_ref):
        a = a0_ref[...] + a1_ref[...]
        t_ref[...] = a[:, :d]
        deg = a[:, d:d + 1]
        norm_ref[...] = 1.0 / jnp.maximum(deg, 1.0)

    return pl.pallas_call(
        body,
        grid=(nb,),
        in_specs=[
            pl.BlockSpec((bn, d_aug), lambda i: (i, 0)),
            pl.BlockSpec((bn, d_aug), lambda i: (i, 0)),
        ],
        out_specs=[
            pl.BlockSpec((bn, d), lambda i: (i, 0)),
            pl.BlockSpec((bn, 1), lambda i: (i, 0)),
        ],
        out_shape=[
            jax.ShapeDtypeStruct((n, d), jnp.float32),
            jax.ShapeDtypeStruct((n, 1), jnp.float32),
        ],
    )


def _make_tc_layer(n, d, bn, relu):
    nb = n // bn

    def body(s0_ref, s1_ref, t_ref, norm_ref, h_ref, wn_ref, wl_ref, out_ref):
        x = (s0_ref[...] + s1_ref[...] - t_ref[...]) * norm_ref[...]
        acc = jnp.dot(x, wn_ref[...], preferred_element_type=jnp.float32,
                      precision=lax.Precision.HIGHEST)
        acc = acc + jnp.dot(h_ref[...], wl_ref[...],
                            preferred_element_type=jnp.float32,
                            precision=lax.Precision.HIGHEST)
        if relu:
            acc = jnp.maximum(acc, 0.0)
        out_ref[...] = acc

    return pl.pallas_call(
        body,
        grid=(nb,),
        in_specs=[
            pl.BlockSpec((bn, d), lambda i: (i, 0)),
            pl.BlockSpec((bn, d), lambda i: (i, 0)),
            pl.BlockSpec((bn, d), lambda i: (i, 0)),
            pl.BlockSpec((bn, 1), lambda i: (i, 0)),
            pl.BlockSpec((bn, d), lambda i: (i, 0)),
            pl.BlockSpec((d, d), lambda i: (0, 0)),
            pl.BlockSpec((d, d), lambda i: (0, 0)),
        ],
        out_specs=pl.BlockSpec((bn, d), lambda i: (i, 0)),
        out_shape=jax.ShapeDtypeStruct((n, d), jnp.float32),
    )


def kernel(features, relations, edge_index, edge_type, W_neigh, W_loop):
    n, d = features.shape
    r = relations.shape[0]
    e = edge_type.shape[0]
    n_layers = W_neigh.shape[0]
    n_workers = _NC * _NS
    n_chunks = e // (n_workers * _K)
    d_aug = d + 16

    src = edge_index[0].reshape(n_workers, n_chunks, _K)
    dst = edge_index[1].reshape(n_workers, n_chunks, _K)
    etype = edge_type.reshape(n_workers, n_chunks, _K)
    aug = jnp.concatenate(
        [relations,
         jnp.ones((r, 1), jnp.float32),
         jnp.zeros((r, d_aug - d - 1), jnp.float32)], axis=1)

    sc_rel, n_pad = _make_sc_segment_sum(r, d_aug, e, n)
    rows_per_tile = n_pad // _NS
    zeros_aug = jnp.zeros((rows_per_tile, d_aug), jnp.float32)
    zeros_d = jnp.zeros((rows_per_tile, d), jnp.float32)
    sc_h, _ = _make_sc_segment_sum(n, d, e, n)
    bn = 2000
    prologue = _make_tc_prologue(n, d_aug, d, bn)

    taug = sc_rel(aug, etype, dst, zeros_aug)
    t_sum, norm = prologue(taug[:n], taug[n_pad:n_pad + n])

    h = features
    for l in range(n_layers):
        s = sc_h(h, src, dst, zeros_d)
        layer = _make_tc_layer(n, d, bn, relu=(l < n_layers - 1))
        h = layer(s[:n], s[n_pad:n_pad + n], t_sum, norm, h,
                  W_neigh[l], W_loop[l])
    return h
